# Initial kernel scaffold; baseline (speedup 1.0000x reference)
import functools

import jax
import jax.numpy as jnp
from jax import lax
from jax.experimental import pallas as pl
from jax.experimental.pallas import tpu as pltpu
from jax.experimental.pallas import tpu_sc as plsc

N = 10000
E = 320000
D = 128

NC = 2
NS = 16
CHUNK = 128
NW = NC * NS
CPT = -(-E // (NW * CHUNK))
EP = CPT * CHUNK * NW
TR = 10016
RPT = TR // NS

_mesh = plsc.VectorSubcoreMesh(core_axis_name="c", subcore_axis_name="s")


@functools.partial(
    pl.kernel,
    out_type=jax.ShapeDtypeStruct((NC, TR, 16), jnp.float32),
    mesh=_mesh,
    scratch_types=[
        pltpu.VMEM((CPT, CHUNK), jnp.int32),
        pltpu.VMEM((CHUNK, 16), jnp.float32),
        pltpu.VMEM_SHARED((TR, 16), jnp.float32),
        pltpu.SemaphoreType.DMA,
    ],
)
def _deg_kernel(dstp_hbm, ones_hbm, zeros_hbm, out_hbm, idx_v, ones_v, deg_sh, sem):
    c = lax.axis_index("c")
    s = lax.axis_index("s")
    wid = c * NS + s
    pltpu.sync_copy(zeros_hbm.at[pl.ds(s * RPT, RPT)], deg_sh.at[pl.ds(s * RPT, RPT)])
    pltpu.sync_copy(dstp_hbm.at[wid], idx_v)
    pltpu.sync_copy(ones_hbm, ones_v)
    plsc.subcore_barrier()

    def body(j, carry):
        pltpu.async_copy(ones_v, deg_sh.at[idx_v.at[j]], sem, add=True).wait()
        return carry

    lax.fori_loop(0, CPT, body, 0)
    plsc.subcore_barrier()
    pltpu.sync_copy(deg_sh.at[pl.ds(s * RPT, RPT)],
                    out_hbm.at[c, pl.ds(s * RPT, RPT)])


@functools.partial(
    pl.kernel,
    out_type=jax.ShapeDtypeStruct((NC, TR, D), jnp.float32),
    mesh=_mesh,
    scratch_types=[
        pltpu.VMEM((CPT, CHUNK), jnp.int32),
        pltpu.VMEM((CPT, CHUNK), jnp.int32),
        pltpu.VMEM((CHUNK, D), jnp.float32),
        pltpu.VMEM_SHARED((TR, D), jnp.float32),
        pltpu.SemaphoreType.DMA,
        pltpu.SemaphoreType.DMA,
    ],
)
def _agg_kernel(hp_hbm, srcp_hbm, dstp_hbm, zeros_hbm, out_hbm,
                idx_s, idx_d, buf, agg_sh, gsem, ssem):
    c = lax.axis_index("c")
    s = lax.axis_index("s")
    wid = c * NS + s
    pltpu.sync_copy(zeros_hbm.at[pl.ds(s * RPT, RPT)], agg_sh.at[pl.ds(s * RPT, RPT)])
    pltpu.sync_copy(srcp_hbm.at[wid], idx_s)
    pltpu.sync_copy(dstp_hbm.at[wid], idx_d)
    plsc.subcore_barrier()

    def body(j, carry):
        pltpu.async_copy(hp_hbm.at[idx_s.at[j]], buf, gsem).wait()
        pltpu.async_copy(buf, agg_sh.at[idx_d.at[j]], ssem, add=True).wait()
        return carry

    lax.fori_loop(0, CPT, body, 0)
    plsc.subcore_barrier()
    pltpu.sync_copy(agg_sh.at[pl.ds(s * RPT, RPT)],
                    out_hbm.at[c, pl.ds(s * RPT, RPT)])


BM = 1000


def _hprime_body(x_ref, w_ref, degp_ref, hp_ref):
    d = degp_ref[0] + degp_ref[1]
    deg = d[:, 0:1] + 1.0
    dinv = lax.rsqrt(deg)
    hp_ref[...] = jnp.dot(x_ref[...] * dinv, w_ref[...],
                          preferred_element_type=jnp.float32)


def _hprime(x, w, degp):
    return pl.pallas_call(
        _hprime_body,
        grid=(N // BM,),
        in_specs=[
            pl.BlockSpec((BM, D), lambda i: (i, 0)),
            pl.BlockSpec((D, D), lambda i: (0, 0)),
            pl.BlockSpec((NC, BM, 16), lambda i: (0, i, 0)),
        ],
        out_specs=pl.BlockSpec((BM, D), lambda i: (i, 0)),
        out_shape=jax.ShapeDtypeStruct((N, D), jnp.float32),
    )(x, w, degp)


def _epilogue_body(x_ref, hp_ref, aggp_ref, degp_ref, b_ref, out_ref):
    d = degp_ref[0] + degp_ref[1]
    deg = d[:, 0:1] + 1.0
    dinv = lax.rsqrt(deg)
    agg = aggp_ref[0] + aggp_ref[1] + hp_ref[...]
    out_ref[...] = x_ref[...] + jnp.maximum(dinv * agg + b_ref[...], 0.0)


def _epilogue(x, hp, aggp, degp, b):
    return pl.pallas_call(
        _epilogue_body,
        grid=(N // BM,),
        in_specs=[
            pl.BlockSpec((BM, D), lambda i: (i, 0)),
            pl.BlockSpec((BM, D), lambda i: (i, 0)),
            pl.BlockSpec((NC, BM, D), lambda i: (0, i, 0)),
            pl.BlockSpec((NC, BM, 16), lambda i: (0, i, 0)),
            pl.BlockSpec((1, D), lambda i: (0, 0)),
        ],
        out_specs=pl.BlockSpec((BM, D), lambda i: (i, 0)),
        out_shape=jax.ShapeDtypeStruct((N, D), jnp.float32),
    )(x, hp, aggp, degp, b)


def kernel(x, edge_index, W, b):
    ei = edge_index.astype(jnp.int32)
    src, dst = ei[0], ei[1]
    pad = EP - E
    pr = jnp.arange(pad, dtype=jnp.int32)
    srcp = jnp.concatenate([src, pr % N]).reshape(NW, CPT, CHUNK)
    dstp = jnp.concatenate([dst, N + pr % (TR - N)]).reshape(NW, CPT, CHUNK)
    zeros16 = jnp.zeros((TR, 16), jnp.float32)
    ones16 = jnp.ones((CHUNK, 16), jnp.float32)
    zerosD = jnp.zeros((TR, D), jnp.float32)

    degp = _deg_kernel(dstp, ones16, zeros16)
    degp_n = degp[:, :N]
    hp = _hprime(x, W, degp_n)
    aggp = _agg_kernel(hp, srcp, dstp, zerosD)
    return _epilogue(x, hp, aggp[:, :N], degp_n, b)


# speedup vs baseline: 22.4828x; 22.4828x over previous
import functools

import jax
import jax.numpy as jnp
from jax import lax
from jax.experimental import pallas as pl
from jax.experimental.pallas import tpu as pltpu
from jax.experimental.pallas import tpu_sc as plsc

N = 10000
E = 320000
D = 128

NC = 2
NS = 16
CHUNK = 128
NW = NC * NS
CPT = -(-E // (NW * CHUNK))
EP = CPT * CHUNK * NW
TR = 10112
RPT = TR // NS

_mesh = plsc.VectorSubcoreMesh(core_axis_name="c", subcore_axis_name="s")


DW = 128
@functools.partial(
    pl.kernel,
    out_type=jax.ShapeDtypeStruct((NC, TR, DW), jnp.float32),
    mesh=_mesh,
    scratch_types=[
        pltpu.VMEM((CPT, CHUNK), jnp.int32),
        pltpu.VMEM((CHUNK, DW), jnp.float32),
        pltpu.VMEM_SHARED((TR, DW), jnp.float32),
        pltpu.SemaphoreType.DMA,
    ],
)
def _deg_kernel(dstp_hbm, ones_hbm, zeros_hbm, out_hbm, idx_v, ones_v, deg_sh, sem):
    c = lax.axis_index("c")
    s = lax.axis_index("s")
    wid = c * NS + s
    pltpu.sync_copy(zeros_hbm.at[pl.ds(s * RPT, RPT)], deg_sh.at[pl.ds(s * RPT, RPT)])
    pltpu.sync_copy(dstp_hbm.at[wid], idx_v)
    pltpu.sync_copy(ones_hbm, ones_v)
    plsc.subcore_barrier()

    def body(j, carry):
        pltpu.async_copy(ones_v, deg_sh.at[idx_v.at[j]], sem, add=True).wait()
        return carry

    lax.fori_loop(0, CPT, body, 0)
    plsc.subcore_barrier()
    pltpu.sync_copy(deg_sh.at[pl.ds(s * RPT, RPT)],
                    out_hbm.at[c, pl.ds(s * RPT, RPT)])


@functools.partial(
    pl.kernel,
    out_type=jax.ShapeDtypeStruct((NC, TR, D), jnp.float32),
    mesh=_mesh,
    scratch_types=[
        pltpu.VMEM((CPT, CHUNK), jnp.int32),
        pltpu.VMEM((CPT, CHUNK), jnp.int32),
        pltpu.VMEM((CHUNK, D), jnp.float32),
        pltpu.VMEM_SHARED((TR, D), jnp.float32),
        pltpu.SemaphoreType.DMA,
        pltpu.SemaphoreType.DMA,
    ],
)
def _agg_kernel(hp_hbm, srcp_hbm, dstp_hbm, zeros_hbm, out_hbm,
                idx_s, idx_d, buf, agg_sh, gsem, ssem):
    c = lax.axis_index("c")
    s = lax.axis_index("s")
    wid = c * NS + s
    pltpu.sync_copy(zeros_hbm.at[pl.ds(s * RPT, RPT)], agg_sh.at[pl.ds(s * RPT, RPT)])
    pltpu.sync_copy(srcp_hbm.at[wid], idx_s)
    pltpu.sync_copy(dstp_hbm.at[wid], idx_d)
    plsc.subcore_barrier()

    def body(j, carry):
        pltpu.async_copy(hp_hbm.at[idx_s.at[j]], buf, gsem).wait()
        pltpu.async_copy(buf, agg_sh.at[idx_d.at[j]], ssem, add=True).wait()
        return carry

    lax.fori_loop(0, CPT, body, 0)
    plsc.subcore_barrier()
    pltpu.sync_copy(agg_sh.at[pl.ds(s * RPT, RPT)],
                    out_hbm.at[c, pl.ds(s * RPT, RPT)])


BM = 1000


def _hprime_body(x_ref, w_ref, degp_ref, hp_ref):
    d = degp_ref[0] + degp_ref[1]
    deg = d[:, 0:1] + 1.0
    dinv = lax.rsqrt(deg)
    hp_ref[...] = jnp.dot(x_ref[...] * dinv, w_ref[...],
                          preferred_element_type=jnp.float32)


def _hprime(x, w, degp):
    return pl.pallas_call(
        _hprime_body,
        grid=(N // BM,),
        in_specs=[
            pl.BlockSpec((BM, D), lambda i: (i, 0)),
            pl.BlockSpec((D, D), lambda i: (0, 0)),
            pl.BlockSpec((NC, BM, DW), lambda i: (0, i, 0)),
        ],
        out_specs=pl.BlockSpec((BM, D), lambda i: (i, 0)),
        out_shape=jax.ShapeDtypeStruct((N, D), jnp.float32),
    )(x, w, degp)


def _epilogue_body(x_ref, hp_ref, aggp_ref, degp_ref, b_ref, out_ref):
    d = degp_ref[0] + degp_ref[1]
    deg = d[:, 0:1] + 1.0
    dinv = lax.rsqrt(deg)
    agg = aggp_ref[0] + aggp_ref[1] + hp_ref[...]
    out_ref[...] = x_ref[...] + jnp.maximum(dinv * agg + b_ref[...], 0.0)


def _epilogue(x, hp, aggp, degp, b):
    return pl.pallas_call(
        _epilogue_body,
        grid=(N // BM,),
        in_specs=[
            pl.BlockSpec((BM, D), lambda i: (i, 0)),
            pl.BlockSpec((BM, D), lambda i: (i, 0)),
            pl.BlockSpec((NC, BM, D), lambda i: (0, i, 0)),
            pl.BlockSpec((NC, BM, DW), lambda i: (0, i, 0)),
            pl.BlockSpec((1, D), lambda i: (0, 0)),
        ],
        out_specs=pl.BlockSpec((BM, D), lambda i: (i, 0)),
        out_shape=jax.ShapeDtypeStruct((N, D), jnp.float32),
    )(x, hp, aggp, degp, b.reshape(1, D))


def kernel(x, edge_index, W, b):
    ei = edge_index.astype(jnp.int32)
    src, dst = ei[0], ei[1]
    pad = EP - E
    pr = jnp.arange(pad, dtype=jnp.int32)
    srcp = jnp.concatenate([src, pr % N]).reshape(NW, CPT, CHUNK)
    dstp = jnp.concatenate([dst, N + pr % (TR - N)]).reshape(NW, CPT, CHUNK)
    onesD = jnp.ones((CHUNK, DW), jnp.float32)
    zerosD = jnp.zeros((TR, D), jnp.float32)

    degp = _deg_kernel(dstp, onesD, zerosD)
    degp_n = degp[:, :N]
    hp = _hprime(x, W, degp_n)
    aggp = _agg_kernel(hp, srcp, dstp, zerosD)
    return _epilogue(x, hp, aggp[:, :N], degp_n, b)


# speedup vs baseline: 24.3812x; 1.0844x over previous
import functools

import jax
import jax.numpy as jnp
from jax import lax
from jax.experimental import pallas as pl
from jax.experimental.pallas import tpu as pltpu
from jax.experimental.pallas import tpu_sc as plsc

N = 10000
E = 320000
D = 128

NC = 2
NS = 16
CHUNK = 128
NW = NC * NS
NBUF = 2
CPT = 80
HPT = CPT // 2
EP = CPT * CHUNK * NW
TR = 10112
RPT = TR // NS
DW = 128

_mesh = plsc.VectorSubcoreMesh(core_axis_name="c", subcore_axis_name="s")


@functools.partial(
    pl.kernel,
    out_type=jax.ShapeDtypeStruct((NC, TR, DW), jnp.float32),
    mesh=_mesh,
    scratch_types=[
        pltpu.VMEM((CPT, CHUNK), jnp.int32),
        pltpu.VMEM((CHUNK, DW), jnp.float32),
        pltpu.VMEM_SHARED((TR, DW), jnp.float32),
        pltpu.SemaphoreType.DMA,
    ],
)
def _deg_kernel(dstp_hbm, ones_hbm, zeros_hbm, out_hbm, idx_v, ones_v, deg_sh, sem):
    c = lax.axis_index("c")
    s = lax.axis_index("s")
    wid = c * NS + s
    pltpu.sync_copy(zeros_hbm.at[pl.ds(s * RPT, RPT)], deg_sh.at[pl.ds(s * RPT, RPT)])
    pltpu.sync_copy(dstp_hbm.at[wid], idx_v)
    pltpu.sync_copy(ones_hbm, ones_v)
    plsc.subcore_barrier()

    def body(i, carry):
        cps = [pltpu.async_copy(ones_v, deg_sh.at[idx_v.at[8 * i + b]], sem,
                                add=True)
               for b in range(8)]
        for cp in cps:
            cp.wait()
        return carry

    lax.fori_loop(0, CPT // 8, body, 0)
    plsc.subcore_barrier()
    pltpu.sync_copy(deg_sh.at[pl.ds(s * RPT, RPT)],
                    out_hbm.at[c, pl.ds(s * RPT, RPT)])


@functools.partial(
    pl.kernel,
    out_type=jax.ShapeDtypeStruct((NC, TR, D), jnp.float32),
    mesh=_mesh,
    scratch_types=[
        pltpu.VMEM((HPT, CHUNK), jnp.int32),
        pltpu.VMEM((HPT, CHUNK), jnp.int32),
        pltpu.VMEM((NBUF, CHUNK, D), jnp.float32),
        pltpu.VMEM_SHARED((TR, D), jnp.float32),
    ] + [pltpu.SemaphoreType.DMA] * (2 * NBUF),
)
def _agg_kernel(hp_hbm, srcp_hbm, dstp_hbm, zeros_hbm, out_hbm,
                idx_s, idx_d, buf, agg_sh, *sems):
    gsems, ssems = sems[:NBUF], sems[NBUF:]
    c = lax.axis_index("c")
    s = lax.axis_index("s")
    wid = c * NS + s
    pltpu.sync_copy(zeros_hbm.at[pl.ds(s * RPT, RPT)], agg_sh.at[pl.ds(s * RPT, RPT)])
    plsc.subcore_barrier()

    def body(i, carry):
        j0 = NBUF * i
        gs = [pltpu.async_copy(hp_hbm.at[idx_s.at[j0 + b]], buf.at[b], gsems[b])
              for b in range(NBUF)]
        ss = []
        for b in range(NBUF):
            gs[b].wait()
            ss.append(pltpu.async_copy(buf.at[b], agg_sh.at[idx_d.at[j0 + b]],
                                       ssems[b], add=True))
        for cp in ss:
            cp.wait()
        return carry

    for p in range(CPT // HPT):
        pltpu.sync_copy(srcp_hbm.at[wid, pl.ds(p * HPT, HPT)], idx_s)
        pltpu.sync_copy(dstp_hbm.at[wid, pl.ds(p * HPT, HPT)], idx_d)
        lax.fori_loop(0, HPT // NBUF, body, 0)
    plsc.subcore_barrier()
    pltpu.sync_copy(agg_sh.at[pl.ds(s * RPT, RPT)],
                    out_hbm.at[c, pl.ds(s * RPT, RPT)])


BM = 1000


def _hprime_body(x_ref, w_ref, degp_ref, hp_ref):
    d = degp_ref[0] + degp_ref[1]
    deg = d[:, 0:1] + 1.0
    dinv = lax.rsqrt(deg)
    hp_ref[...] = jnp.dot(x_ref[...] * dinv, w_ref[...],
                          preferred_element_type=jnp.float32)


def _hprime(x, w, degp):
    return pl.pallas_call(
        _hprime_body,
        grid=(N // BM,),
        in_specs=[
            pl.BlockSpec((BM, D), lambda i: (i, 0)),
            pl.BlockSpec((D, D), lambda i: (0, 0)),
            pl.BlockSpec((NC, BM, DW), lambda i: (0, i, 0)),
        ],
        out_specs=pl.BlockSpec((BM, D), lambda i: (i, 0)),
        out_shape=jax.ShapeDtypeStruct((N, D), jnp.float32),
    )(x, w, degp)


def _epilogue_body(x_ref, hp_ref, aggp_ref, degp_ref, b_ref, out_ref):
    d = degp_ref[0] + degp_ref[1]
    deg = d[:, 0:1] + 1.0
    dinv = lax.rsqrt(deg)
    agg = aggp_ref[0] + aggp_ref[1] + hp_ref[...]
    out_ref[...] = x_ref[...] + jnp.maximum(dinv * agg + b_ref[...], 0.0)


def _epilogue(x, hp, aggp, degp, b):
    return pl.pallas_call(
        _epilogue_body,
        grid=(N // BM,),
        in_specs=[
            pl.BlockSpec((BM, D), lambda i: (i, 0)),
            pl.BlockSpec((BM, D), lambda i: (i, 0)),
            pl.BlockSpec((NC, BM, D), lambda i: (0, i, 0)),
            pl.BlockSpec((NC, BM, DW), lambda i: (0, i, 0)),
            pl.BlockSpec((1, D), lambda i: (0, 0)),
        ],
        out_specs=pl.BlockSpec((BM, D), lambda i: (i, 0)),
        out_shape=jax.ShapeDtypeStruct((N, D), jnp.float32),
    )(x, hp, aggp, degp, b.reshape(1, D))


def kernel(x, edge_index, W, b):
    ei = edge_index.astype(jnp.int32)
    src, dst = ei[0], ei[1]
    pad = EP - E
    pr = jnp.arange(pad, dtype=jnp.int32)
    srcp = jnp.concatenate([src, pr % N]).reshape(NW, CPT, CHUNK)
    dstp = jnp.concatenate([dst, N + pr % (TR - N)]).reshape(NW, CPT, CHUNK)
    onesD = jnp.ones((CHUNK, DW), jnp.float32)
    zerosD = jnp.zeros((TR, D), jnp.float32)

    degp = _deg_kernel(dstp, onesD, zerosD)
    degp_n = degp[:, :N]
    hp = _hprime(x, W, degp_n)
    aggp = _agg_kernel(hp, srcp, dstp, zerosD)
    return _epilogue(x, hp, aggp[:, :N], degp_n, b)


# speedup vs baseline: 24.6095x; 1.0094x over previous
import functools

import jax
import jax.numpy as jnp
from jax import lax
from jax.experimental import pallas as pl
from jax.experimental.pallas import tpu as pltpu
from jax.experimental.pallas import tpu_sc as plsc

N = 10000
E = 320000
D = 128

NC = 2
NS = 16
CHUNK = 128
NW = NC * NS
NBUF = 2
CPT = 80
HPT = CPT // 2
EP = CPT * CHUNK * NW
TR = 10112
RPT = TR // NS
DW = 128

_mesh = plsc.VectorSubcoreMesh(core_axis_name="c", subcore_axis_name="s")


@functools.partial(
    pl.kernel,
    out_type=jax.ShapeDtypeStruct((NC, TR, DW), jnp.float32),
    mesh=_mesh,
    scratch_types=[
        pltpu.VMEM((CPT, CHUNK), jnp.int32),
        pltpu.VMEM((CHUNK, DW), jnp.float32),
        pltpu.VMEM_SHARED((TR, DW), jnp.float32),
        pltpu.SemaphoreType.DMA,
    ],
)
def _deg_kernel(dstp_hbm, ones_hbm, zeros_hbm, out_hbm, idx_v, ones_v, deg_sh, sem):
    c = lax.axis_index("c")
    s = lax.axis_index("s")
    wid = c * NS + s
    pltpu.sync_copy(zeros_hbm.at[pl.ds(s * RPT, RPT)], deg_sh.at[pl.ds(s * RPT, RPT)])
    pltpu.sync_copy(dstp_hbm.at[wid], idx_v)
    pltpu.sync_copy(ones_hbm, ones_v)
    plsc.subcore_barrier()

    def body(i, carry):
        cps = [pltpu.async_copy(ones_v, deg_sh.at[idx_v.at[8 * i + b]], sem,
                                add=True)
               for b in range(8)]
        for cp in cps:
            cp.wait()
        return carry

    lax.fori_loop(0, CPT // 8, body, 0)
    plsc.subcore_barrier()
    pltpu.sync_copy(deg_sh.at[pl.ds(s * RPT, RPT)],
                    out_hbm.at[c, pl.ds(s * RPT, RPT)])


@functools.partial(
    pl.kernel,
    out_type=jax.ShapeDtypeStruct((NC, TR, D), jnp.float32),
    mesh=_mesh,
    scratch_types=[
        pltpu.VMEM((HPT, CHUNK), jnp.int32),
        pltpu.VMEM((HPT, CHUNK), jnp.int32),
        pltpu.VMEM((NBUF, CHUNK, D), jnp.float32),
        pltpu.VMEM_SHARED((TR, D), jnp.float32),
    ] + [pltpu.SemaphoreType.DMA] * (2 * NBUF),
)
def _agg_kernel(hp_hbm, srcp_hbm, dstp_hbm, zeros_hbm, out_hbm,
                idx_s, idx_d, buf, agg_sh, *sems):
    gsems, ssems = sems[:NBUF], sems[NBUF:]
    c = lax.axis_index("c")
    s = lax.axis_index("s")
    wid = c * NS + s
    pltpu.sync_copy(zeros_hbm.at[pl.ds(s * RPT, RPT)], agg_sh.at[pl.ds(s * RPT, RPT)])
    plsc.subcore_barrier()

    def _gwait(b, j):
        pltpu.make_async_copy(hp_hbm.at[idx_s.at[j]], buf.at[b], gsems[b]).wait()

    def _sstart(b, j):
        pltpu.async_copy(buf.at[b], agg_sh.at[idx_d.at[j]], ssems[b], add=True)

    def _swait(b, j):
        pltpu.make_async_copy(buf.at[b], agg_sh.at[idx_d.at[j]], ssems[b]).wait()

    def _gstart(b, j):
        pltpu.async_copy(hp_hbm.at[idx_s.at[j]], buf.at[b], gsems[b])

    for p in range(CPT // HPT):
        pltpu.sync_copy(srcp_hbm.at[wid, pl.ds(p * HPT, HPT)], idx_s)
        pltpu.sync_copy(dstp_hbm.at[wid, pl.ds(p * HPT, HPT)], idx_d)
        for b in range(NBUF):
            _gstart(b, b)

        def body(i, carry):
            for b in range(NBUF):
                j = NBUF * i + b
                _gwait(b, j)
                _sstart(b, j)
            for b in range(NBUF):
                j = NBUF * i + b
                _swait(b, j)
                _gstart(b, j + NBUF)
            return carry

        lax.fori_loop(0, HPT // NBUF - 1, body, 0)
        for b in range(NBUF):
            j = HPT - NBUF + b
            _gwait(b, j)
            _sstart(b, j)
        for b in range(NBUF):
            j = HPT - NBUF + b
            _swait(b, j)
    plsc.subcore_barrier()
    pltpu.sync_copy(agg_sh.at[pl.ds(s * RPT, RPT)],
                    out_hbm.at[c, pl.ds(s * RPT, RPT)])


BM = 1000


def _hprime_body(x_ref, w_ref, degp_ref, hp_ref):
    d = degp_ref[0] + degp_ref[1]
    deg = d[:, 0:1] + 1.0
    dinv = lax.rsqrt(deg)
    hp_ref[...] = jnp.dot(x_ref[...] * dinv, w_ref[...],
                          preferred_element_type=jnp.float32)


def _hprime(x, w, degp):
    return pl.pallas_call(
        _hprime_body,
        grid=(N // BM,),
        in_specs=[
            pl.BlockSpec((BM, D), lambda i: (i, 0)),
            pl.BlockSpec((D, D), lambda i: (0, 0)),
            pl.BlockSpec((NC, BM, DW), lambda i: (0, i, 0)),
        ],
        out_specs=pl.BlockSpec((BM, D), lambda i: (i, 0)),
        out_shape=jax.ShapeDtypeStruct((N, D), jnp.float32),
    )(x, w, degp)


def _epilogue_body(x_ref, hp_ref, aggp_ref, degp_ref, b_ref, out_ref):
    d = degp_ref[0] + degp_ref[1]
    deg = d[:, 0:1] + 1.0
    dinv = lax.rsqrt(deg)
    agg = aggp_ref[0] + aggp_ref[1] + hp_ref[...]
    out_ref[...] = x_ref[...] + jnp.maximum(dinv * agg + b_ref[...], 0.0)


def _epilogue(x, hp, aggp, degp, b):
    return pl.pallas_call(
        _epilogue_body,
        grid=(N // BM,),
        in_specs=[
            pl.BlockSpec((BM, D), lambda i: (i, 0)),
            pl.BlockSpec((BM, D), lambda i: (i, 0)),
            pl.BlockSpec((NC, BM, D), lambda i: (0, i, 0)),
            pl.BlockSpec((NC, BM, DW), lambda i: (0, i, 0)),
            pl.BlockSpec((1, D), lambda i: (0, 0)),
        ],
        out_specs=pl.BlockSpec((BM, D), lambda i: (i, 0)),
        out_shape=jax.ShapeDtypeStruct((N, D), jnp.float32),
    )(x, hp, aggp, degp, b.reshape(1, D))


def kernel(x, edge_index, W, b):
    ei = edge_index.astype(jnp.int32)
    src, dst = ei[0], ei[1]
    pad = EP - E
    pr = jnp.arange(pad, dtype=jnp.int32)
    srcp = jnp.concatenate([src, pr % N]).reshape(NW, CPT, CHUNK)
    dstp = jnp.concatenate([dst, N + pr % (TR - N)]).reshape(NW, CPT, CHUNK)
    onesD = jnp.ones((CHUNK, DW), jnp.float32)
    zerosD = jnp.zeros((TR, D), jnp.float32)

    degp = _deg_kernel(dstp, onesD, zerosD)
    degp_n = degp[:, :N]
    hp = _hprime(x, W, degp_n)
    aggp = _agg_kernel(hp, srcp, dstp, zerosD)
    return _epilogue(x, hp, aggp[:, :N], degp_n, b)


# speedup vs baseline: 31.1344x; 1.2651x over previous
import functools

import jax
import jax.numpy as jnp
from jax import lax
from jax.experimental import pallas as pl
from jax.experimental.pallas import tpu as pltpu
from jax.experimental.pallas import tpu_sc as plsc

N = 10000
E = 320000
D = 128

NC = 2
NS = 16
CHUNK = 128
NW = NC * NS
NBUF = 2
CPT = 80
HPT = CPT // 2
EP = CPT * CHUNK * NW
TR = 10240
RPT = TR // NS

_mesh = plsc.VectorSubcoreMesh(core_axis_name="c", subcore_axis_name="s")


@functools.partial(
    pl.kernel,
    out_type=jax.ShapeDtypeStruct((NW, TR), jnp.float32),
    mesh=_mesh,
    compiler_params=pltpu.CompilerParams(needs_layout_passes=False),
    scratch_types=[
        pltpu.VMEM((CPT, CHUNK), jnp.int32),
        pltpu.VMEM((TR,), jnp.float32),
    ],
)
def _deg_kernel(dstp_hbm, out_hbm, idx_v, hist):
    c = lax.axis_index("c")
    s = lax.axis_index("s")
    wid = c * NS + s
    pltpu.sync_copy(dstp_hbm.at[wid], idx_v)

    def zbody(i, carry):
        hist[pl.ds(i * 16, 16)] = jnp.zeros((16,), jnp.float32)
        return carry

    lax.fori_loop(0, TR // 16, zbody, 0)

    def body(j, carry):
        for k in range(CHUNK // 16):
            x = idx_v[j, pl.ds(k * 16, 16)]
            cnt, last = plsc.scan_count(x)
            plsc.addupdate_scatter(hist, [x], cnt.astype(jnp.float32), mask=last)
        return carry

    lax.fori_loop(0, CPT, body, 0)
    pltpu.sync_copy(hist, out_hbm.at[wid])


@functools.partial(
    pl.kernel,
    out_type=jax.ShapeDtypeStruct((NC, TR, D), jnp.float32),
    mesh=_mesh,
    scratch_types=[
        pltpu.VMEM((HPT, CHUNK), jnp.int32),
        pltpu.VMEM((HPT, CHUNK), jnp.int32),
        pltpu.VMEM((NBUF, CHUNK, D), jnp.float32),
        pltpu.VMEM_SHARED((TR, D), jnp.float32),
    ] + [pltpu.SemaphoreType.DMA] * (2 * NBUF),
)
def _agg_kernel(hp_hbm, srcp_hbm, dstp_hbm, zeros_hbm, out_hbm,
                idx_s, idx_d, buf, agg_sh, *sems):
    gsems, ssems = sems[:NBUF], sems[NBUF:]
    c = lax.axis_index("c")
    s = lax.axis_index("s")
    wid = c * NS + s
    pltpu.sync_copy(zeros_hbm.at[pl.ds(s * RPT, RPT)], agg_sh.at[pl.ds(s * RPT, RPT)])
    plsc.subcore_barrier()

    def _gwait(b, j):
        pltpu.make_async_copy(hp_hbm.at[idx_s.at[j]], buf.at[b], gsems[b]).wait()

    def _sstart(b, j):
        pltpu.async_copy(buf.at[b], agg_sh.at[idx_d.at[j]], ssems[b], add=True)

    def _swait(b, j):
        pltpu.make_async_copy(buf.at[b], agg_sh.at[idx_d.at[j]], ssems[b]).wait()

    def _gstart(b, j):
        pltpu.async_copy(hp_hbm.at[idx_s.at[j]], buf.at[b], gsems[b])

    for p in range(CPT // HPT):
        pltpu.sync_copy(srcp_hbm.at[wid, pl.ds(p * HPT, HPT)], idx_s)
        pltpu.sync_copy(dstp_hbm.at[wid, pl.ds(p * HPT, HPT)], idx_d)
        for b in range(NBUF):
            _gstart(b, b)

        def body(i, carry):
            for b in range(NBUF):
                j = NBUF * i + b
                _gwait(b, j)
                _sstart(b, j)
            for b in range(NBUF):
                j = NBUF * i + b
                _swait(b, j)
                _gstart(b, j + NBUF)
            return carry

        lax.fori_loop(0, HPT // NBUF - 1, body, 0)
        for b in range(NBUF):
            j = HPT - NBUF + b
            _gwait(b, j)
            _sstart(b, j)
        for b in range(NBUF):
            j = HPT - NBUF + b
            _swait(b, j)
    plsc.subcore_barrier()
    pltpu.sync_copy(agg_sh.at[pl.ds(s * RPT, RPT)],
                    out_hbm.at[c, pl.ds(s * RPT, RPT)])


BM = 1024


def _hprime_body(x_ref, w_ref, degc_ref, hp_ref):
    dinv = lax.rsqrt(degc_ref[...] + 1.0)
    hp_ref[...] = jnp.dot(x_ref[...] * dinv, w_ref[...],
                          preferred_element_type=jnp.float32)


def _hprime(x, w, degc):
    return pl.pallas_call(
        _hprime_body,
        grid=(TR // BM,),
        in_specs=[
            pl.BlockSpec((BM, D), lambda i: (i, 0)),
            pl.BlockSpec((D, D), lambda i: (0, 0)),
            pl.BlockSpec((BM, 1), lambda i: (i, 0)),
        ],
        out_specs=pl.BlockSpec((BM, D), lambda i: (i, 0)),
        out_shape=jax.ShapeDtypeStruct((TR, D), jnp.float32),
    )(x, w, degc)


def _epilogue_body(x_ref, hp_ref, aggp_ref, degc_ref, b_ref, out_ref):
    dinv = lax.rsqrt(degc_ref[...] + 1.0)
    agg = aggp_ref[0] + aggp_ref[1] + hp_ref[...]
    out_ref[...] = x_ref[...] + jnp.maximum(dinv * agg + b_ref[...], 0.0)


def _epilogue(x, hp, aggp, degc, b):
    return pl.pallas_call(
        _epilogue_body,
        grid=(TR // BM,),
        in_specs=[
            pl.BlockSpec((BM, D), lambda i: (i, 0)),
            pl.BlockSpec((BM, D), lambda i: (i, 0)),
            pl.BlockSpec((NC, BM, D), lambda i: (0, i, 0)),
            pl.BlockSpec((BM, 1), lambda i: (i, 0)),
            pl.BlockSpec((1, D), lambda i: (0, 0)),
        ],
        out_specs=pl.BlockSpec((BM, D), lambda i: (i, 0)),
        out_shape=jax.ShapeDtypeStruct((TR, D), jnp.float32),
    )(x, hp, aggp, degc, b.reshape(1, D))


def kernel(x, edge_index, W, b):
    ei = edge_index.astype(jnp.int32)
    src, dst = ei[0], ei[1]
    pad = EP - E
    pr = jnp.arange(pad, dtype=jnp.int32)
    srcp = jnp.concatenate([src, pr % N]).reshape(NW, CPT, CHUNK)
    dstp = jnp.concatenate([dst, N + pr % (TR - N)]).reshape(NW, CPT, CHUNK)
    zerosD = jnp.zeros((TR, D), jnp.float32)
    x_pad = jnp.concatenate([x, jnp.zeros((TR - N, D), jnp.float32)])

    degp = _deg_kernel(dstp)
    degc = jnp.sum(degp, axis=0).reshape(TR, 1)
    hp = _hprime(x_pad, W, degc)
    aggp = _agg_kernel(hp, srcp, dstp, zerosD)
    return _epilogue(x_pad, hp, aggp, degc, b)[:N]


# speedup vs baseline: 32.0562x; 1.0296x over previous
import functools

import jax
import jax.numpy as jnp
from jax import lax
from jax.experimental import pallas as pl
from jax.experimental.pallas import tpu as pltpu
from jax.experimental.pallas import tpu_sc as plsc

N = 10000
E = 320000
D = 128

NC = 2
NS = 16
CHUNK = 128
NW = NC * NS
NBUF = 2
CPT = 80
HPT = CPT // 2
EP = CPT * CHUNK * NW
TR = 10240
RPT = TR // NS

_mesh = plsc.VectorSubcoreMesh(core_axis_name="c", subcore_axis_name="s")


@functools.partial(
    pl.kernel,
    out_type=jax.ShapeDtypeStruct((NW, TR), jnp.float32),
    mesh=_mesh,
    compiler_params=pltpu.CompilerParams(needs_layout_passes=False),
    scratch_types=[
        pltpu.VMEM((CPT, CHUNK), jnp.int32),
        pltpu.VMEM((TR,), jnp.float32),
    ],
)
def _deg_kernel(dstp_hbm, out_hbm, idx_v, hist):
    c = lax.axis_index("c")
    s = lax.axis_index("s")
    wid = c * NS + s
    pltpu.sync_copy(dstp_hbm.at[wid], idx_v)

    def zbody(i, carry):
        hist[pl.ds(i * 16, 16)] = jnp.zeros((16,), jnp.float32)
        return carry

    lax.fori_loop(0, TR // 16, zbody, 0)

    def body(j, carry):
        for k in range(CHUNK // 16):
            x = idx_v[j, pl.ds(k * 16, 16)]
            cnt, last = plsc.scan_count(x)
            plsc.addupdate_scatter(hist, [x], cnt.astype(jnp.float32), mask=last)
        return carry

    lax.fori_loop(0, CPT, body, 0)
    pltpu.sync_copy(hist, out_hbm.at[wid])


@functools.partial(
    pl.kernel,
    out_type=jax.ShapeDtypeStruct((NC, TR, D), jnp.float32),
    mesh=_mesh,
    scratch_types=[
        pltpu.VMEM((HPT, CHUNK), jnp.int32),
        pltpu.VMEM((HPT, CHUNK), jnp.int32),
        pltpu.VMEM((NBUF, CHUNK, D), jnp.float32),
        pltpu.VMEM_SHARED((TR, D), jnp.float32),
    ] + [pltpu.SemaphoreType.DMA] * (2 * NBUF),
)
def _agg_kernel(hp_hbm, srcp_hbm, dstp_hbm, zeros_hbm, out_hbm,
                idx_s, idx_d, buf, agg_sh, *sems):
    gsems, ssems = sems[:NBUF], sems[NBUF:]
    c = lax.axis_index("c")
    s = lax.axis_index("s")
    wid = c * NS + s
    pltpu.sync_copy(zeros_hbm.at[pl.ds(s * RPT, RPT)], agg_sh.at[pl.ds(s * RPT, RPT)])
    plsc.subcore_barrier()

    def _gwait(b, j):
        pltpu.make_async_copy(hp_hbm.at[idx_s.at[j]], buf.at[b], gsems[b]).wait()

    def _sstart(b, j):
        pltpu.async_copy(buf.at[b], agg_sh.at[idx_d.at[j]], ssems[b], add=True)

    def _swait(b, j):
        pltpu.make_async_copy(buf.at[b], agg_sh.at[idx_d.at[j]], ssems[b]).wait()

    def _gstart(b, j):
        pltpu.async_copy(hp_hbm.at[idx_s.at[j]], buf.at[b], gsems[b])

    for p in range(CPT // HPT):
        pltpu.sync_copy(srcp_hbm.at[wid, pl.ds(p * HPT, HPT)], idx_s)
        pltpu.sync_copy(dstp_hbm.at[wid, pl.ds(p * HPT, HPT)], idx_d)
        for b in range(NBUF):
            _gstart(b, b)

        def body(i, carry):
            for b in range(NBUF):
                j = NBUF * i + b
                _gwait(b, j)
                _sstart(b, j)
            for b in range(NBUF):
                j = NBUF * i + b
                _swait(b, j)
                _gstart(b, j + NBUF)
            return carry

        lax.fori_loop(0, HPT // NBUF - 1, body, 0)
        for b in range(NBUF):
            j = HPT - NBUF + b
            _gwait(b, j)
            _sstart(b, j)
        for b in range(NBUF):
            j = HPT - NBUF + b
            _swait(b, j)
    plsc.subcore_barrier()
    pltpu.sync_copy(agg_sh.at[pl.ds(s * RPT, RPT)],
                    out_hbm.at[c, pl.ds(s * RPT, RPT)])


BM = 1000


def _hprime_body(x_ref, w_ref, degc_ref, hp_ref):
    dinv = lax.rsqrt(degc_ref[...] + 1.0)
    hp_ref[...] = jnp.dot(x_ref[...] * dinv, w_ref[...],
                          preferred_element_type=jnp.float32)


def _hprime(x, w, degc):
    return pl.pallas_call(
        _hprime_body,
        grid=(N // BM,),
        in_specs=[
            pl.BlockSpec((BM, D), lambda i: (i, 0)),
            pl.BlockSpec((D, D), lambda i: (0, 0)),
            pl.BlockSpec((BM, 1), lambda i: (i, 0)),
        ],
        out_specs=pl.BlockSpec((BM, D), lambda i: (i, 0)),
        out_shape=jax.ShapeDtypeStruct((N, D), jnp.float32),
    )(x, w, degc)


def _epilogue_body(x_ref, hp_ref, aggp_ref, degc_ref, b_ref, out_ref):
    dinv = lax.rsqrt(degc_ref[...] + 1.0)
    agg = aggp_ref[0] + aggp_ref[1] + hp_ref[...]
    out_ref[...] = x_ref[...] + jnp.maximum(dinv * agg + b_ref[...], 0.0)


def _epilogue(x, hp, aggp, degc, b):
    return pl.pallas_call(
        _epilogue_body,
        grid=(N // BM,),
        in_specs=[
            pl.BlockSpec((BM, D), lambda i: (i, 0)),
            pl.BlockSpec((BM, D), lambda i: (i, 0)),
            pl.BlockSpec((NC, BM, D), lambda i: (0, i, 0)),
            pl.BlockSpec((BM, 1), lambda i: (i, 0)),
            pl.BlockSpec((1, D), lambda i: (0, 0)),
        ],
        out_specs=pl.BlockSpec((BM, D), lambda i: (i, 0)),
        out_shape=jax.ShapeDtypeStruct((N, D), jnp.float32),
    )(x, hp, aggp, degc, b.reshape(1, D))


def kernel(x, edge_index, W, b):
    ei = edge_index.astype(jnp.int32)
    src, dst = ei[0], ei[1]
    pad = EP - E
    pr = jnp.arange(pad, dtype=jnp.int32)
    srcp = jnp.concatenate([src, pr % N]).reshape(NW, CPT, CHUNK)
    dstp = jnp.concatenate([dst, N + pr % (TR - N)]).reshape(NW, CPT, CHUNK)
    zerosD = jnp.zeros((TR, D), jnp.float32)

    degp = _deg_kernel(dstp)
    degc = jnp.sum(degp, axis=0).reshape(TR, 1)
    hp = _hprime(x, W, degc)
    aggp = _agg_kernel(hp, srcp, dstp, zerosD)
    return _epilogue(x, hp, aggp, degc, b)


# speedup vs baseline: 32.6694x; 1.0191x over previous
import functools

import jax
import jax.numpy as jnp
from jax import lax
from jax.experimental import pallas as pl
from jax.experimental.pallas import tpu as pltpu
from jax.experimental.pallas import tpu_sc as plsc

N = 10000
E = 320000
D = 128

NC = 2
NS = 16
CHUNK = 128
NW = NC * NS
NBUF = 2
CPT = 80
HPT = CPT // 2
EP = CPT * CHUNK * NW
TR = 10240
RPT = TR // NS

_mesh = plsc.VectorSubcoreMesh(core_axis_name="c", subcore_axis_name="s")


EPW = E // NW
EPWP = CPT * CHUNK


@functools.partial(
    pl.kernel,
    out_type=jax.ShapeDtypeStruct((NW, TR), jnp.float32),
    mesh=_mesh,
    compiler_params=pltpu.CompilerParams(needs_layout_passes=False),
    scratch_types=[
        pltpu.VMEM((EPWP,), jnp.int32),
        pltpu.VMEM((TR,), jnp.float32),
    ],
)
def _deg_kernel(dst_hbm, out_hbm, idx_v, hist):
    c = lax.axis_index("c")
    s = lax.axis_index("s")
    wid = c * NS + s
    pltpu.sync_copy(dst_hbm.at[pl.ds(wid * EPW, EPW)], idx_v.at[pl.ds(0, EPW)])
    for k in range((EPWP - EPW) // 16):
        idx_v[pl.ds(EPW + 16 * k, 16)] = N + 16 * k + lax.iota(jnp.int32, 16)

    def zbody(i, carry):
        hist[pl.ds(i * 16, 16)] = jnp.zeros((16,), jnp.float32)
        return carry

    lax.fori_loop(0, TR // 16, zbody, 0)

    def body(j, carry):
        x = idx_v[pl.ds(j * 16, 16)]
        cnt, last = plsc.scan_count(x)
        plsc.addupdate_scatter(hist, [x], cnt.astype(jnp.float32), mask=last)
        return carry

    lax.fori_loop(0, EPWP // 16, body, 0)
    pltpu.sync_copy(hist, out_hbm.at[wid])


@functools.partial(
    pl.kernel,
    out_type=jax.ShapeDtypeStruct((NC, TR, D), jnp.float32),
    mesh=_mesh,
    scratch_types=[
        pltpu.VMEM((EPWP,), jnp.int32),
        pltpu.VMEM((HPT, CHUNK), jnp.int32),
        pltpu.VMEM((NBUF, CHUNK, D), jnp.float32),
        pltpu.VMEM_SHARED((TR, D), jnp.float32),
    ] + [pltpu.SemaphoreType.DMA] * (2 * NBUF),
)
def _agg_kernel(hp_hbm, src_hbm, dstp_hbm, out_hbm,
                idx_s, idx_d, buf, agg_sh, *sems):
    gsems, ssems = sems[:NBUF], sems[NBUF:]
    c = lax.axis_index("c")
    s = lax.axis_index("s")
    wid = c * NS + s
    pltpu.sync_copy(src_hbm.at[pl.ds(wid * EPW, EPW)], idx_s.at[pl.ds(0, EPW)])
    for k in range((EPWP - EPW) // 16):
        idx_s[pl.ds(EPW + 16 * k, 16)] = wid * 256 + 16 * k + lax.iota(jnp.int32, 16)

    def zrow(r, carry):
        for k in range(D // 16):
            buf[0, r, pl.ds(k * 16, 16)] = jnp.zeros((16,), jnp.float32)
        return carry

    lax.fori_loop(0, CHUNK, zrow, 0)
    for q in range(RPT // CHUNK):
        pltpu.sync_copy(buf.at[0], agg_sh.at[pl.ds(s * RPT + q * CHUNK, CHUNK)])
    plsc.subcore_barrier()

    def _gwait(b, j):
        pltpu.make_async_copy(hp_hbm.at[idx_s.at[pl.ds(j * CHUNK, CHUNK)]],
                              buf.at[b], gsems[b]).wait()

    def _sstart(b, j):
        pltpu.async_copy(buf.at[b], agg_sh.at[idx_d.at[j]], ssems[b], add=True)

    def _swait(b, j):
        pltpu.make_async_copy(buf.at[b], agg_sh.at[idx_d.at[j]], ssems[b]).wait()

    def _gstart(b, j):
        pltpu.async_copy(hp_hbm.at[idx_s.at[pl.ds(j * CHUNK, CHUNK)]],
                         buf.at[b], gsems[b])

    for p in range(CPT // HPT):
        j0 = p * HPT
        pltpu.sync_copy(dstp_hbm.at[wid, pl.ds(p * HPT, HPT)], idx_d)
        for b in range(NBUF):
            _gstart(b, j0 + b)

        def body(i, carry):
            for b in range(NBUF):
                j = NBUF * i + b
                _gwait(b, j0 + j)
                _sstart(b, j)
            for b in range(NBUF):
                j = NBUF * i + b
                _swait(b, j)
                _gstart(b, j0 + j + NBUF)
            return carry

        lax.fori_loop(0, HPT // NBUF - 1, body, 0)
        for b in range(NBUF):
            j = HPT - NBUF + b
            _gwait(b, j0 + j)
            _sstart(b, j)
        for b in range(NBUF):
            j = HPT - NBUF + b
            _swait(b, j)
    plsc.subcore_barrier()
    pltpu.sync_copy(agg_sh.at[pl.ds(s * RPT, RPT)],
                    out_hbm.at[c, pl.ds(s * RPT, RPT)])


BM = 1000


def _hprime_body(x_ref, w_ref, degc_ref, hp_ref):
    dinv = lax.rsqrt(degc_ref[...] + 1.0)
    hp_ref[...] = jnp.dot(x_ref[...] * dinv, w_ref[...],
                          preferred_element_type=jnp.float32)


def _hprime(x, w, degc):
    return pl.pallas_call(
        _hprime_body,
        grid=(N // BM,),
        in_specs=[
            pl.BlockSpec((BM, D), lambda i: (i, 0)),
            pl.BlockSpec((D, D), lambda i: (0, 0)),
            pl.BlockSpec((BM, 1), lambda i: (i, 0)),
        ],
        out_specs=pl.BlockSpec((BM, D), lambda i: (i, 0)),
        out_shape=jax.ShapeDtypeStruct((N, D), jnp.float32),
    )(x, w, degc)


def _epilogue_body(x_ref, hp_ref, aggp_ref, degc_ref, b_ref, out_ref):
    dinv = lax.rsqrt(degc_ref[...] + 1.0)
    agg = aggp_ref[0] + aggp_ref[1] + hp_ref[...]
    out_ref[...] = x_ref[...] + jnp.maximum(dinv * agg + b_ref[...], 0.0)


def _epilogue(x, hp, aggp, degc, b):
    return pl.pallas_call(
        _epilogue_body,
        grid=(N // BM,),
        in_specs=[
            pl.BlockSpec((BM, D), lambda i: (i, 0)),
            pl.BlockSpec((BM, D), lambda i: (i, 0)),
            pl.BlockSpec((NC, BM, D), lambda i: (0, i, 0)),
            pl.BlockSpec((BM, 1), lambda i: (i, 0)),
            pl.BlockSpec((1, D), lambda i: (0, 0)),
        ],
        out_specs=pl.BlockSpec((BM, D), lambda i: (i, 0)),
        out_shape=jax.ShapeDtypeStruct((N, D), jnp.float32),
    )(x, hp, aggp, degc, b.reshape(1, D))


def kernel(x, edge_index, W, b):
    ei = edge_index.astype(jnp.int32)
    src, dst = ei[0], ei[1]
    padd = jnp.broadcast_to(N + jnp.arange(EPWP - EPW, dtype=jnp.int32),
                            (NW, EPWP - EPW))
    dstp = jnp.concatenate([dst.reshape(NW, EPW), padd], axis=1)
    dstp = dstp.reshape(NW, CPT, CHUNK)

    degp = _deg_kernel(dst)
    degc = jnp.sum(degp, axis=0).reshape(TR, 1)
    hp = _hprime(x, W, degc)
    aggp = _agg_kernel(hp, src, dstp)
    return _epilogue(x, hp, aggp, degc, b)


# speedup vs baseline: 34.4657x; 1.0550x over previous
import functools

import jax
import jax.numpy as jnp
from jax import lax
from jax.experimental import pallas as pl
from jax.experimental.pallas import tpu as pltpu
from jax.experimental.pallas import tpu_sc as plsc

N = 10000
E = 320000
D = 128

NC = 2
NS = 16
CHUNK = 128
NW = NC * NS
NBUF = 2
CPT = 80
HPT = CPT // 2
EP = CPT * CHUNK * NW
TR = 10240
RPT = TR // NS

_mesh = plsc.VectorSubcoreMesh(core_axis_name="c", subcore_axis_name="s")


EPW = E // NW
EPWP = CPT * CHUNK


@functools.partial(
    pl.kernel,
    out_type=jax.ShapeDtypeStruct((NW, TR), jnp.float32),
    mesh=_mesh,
    compiler_params=pltpu.CompilerParams(needs_layout_passes=False),
    scratch_types=[
        pltpu.VMEM((EPWP,), jnp.int32),
        pltpu.VMEM((TR,), jnp.float32),
    ],
)
def _deg_kernel(eif_hbm, out_hbm, idx_v, hist):
    c = lax.axis_index("c")
    s = lax.axis_index("s")
    wid = c * NS + s
    pltpu.sync_copy(eif_hbm.at[pl.ds(E + wid * EPW, EPW)], idx_v.at[pl.ds(0, EPW)])
    for k in range((EPWP - EPW) // 16):
        idx_v[pl.ds(EPW + 16 * k, 16)] = N + 16 * k + lax.iota(jnp.int32, 16)

    def zbody(i, carry):
        hist[pl.ds(i * 16, 16)] = jnp.zeros((16,), jnp.float32)
        return carry

    lax.fori_loop(0, TR // 16, zbody, 0)

    def body(j, carry):
        x = idx_v[pl.ds(j * 16, 16)]
        cnt, last = plsc.scan_count(x)
        plsc.addupdate_scatter(hist, [x], cnt.astype(jnp.float32), mask=last)
        return carry

    lax.fori_loop(0, EPWP // 16, body, 0)
    pltpu.sync_copy(hist, out_hbm.at[wid])


@functools.partial(
    pl.kernel,
    out_type=jax.ShapeDtypeStruct((NC, TR, D), jnp.float32),
    mesh=_mesh,
    scratch_types=[
        pltpu.VMEM((EPWP,), jnp.int32),
        pltpu.VMEM((HPT, CHUNK), jnp.int32),
        pltpu.VMEM((NBUF, CHUNK, D), jnp.float32),
        pltpu.VMEM_SHARED((TR, D), jnp.float32),
    ] + [pltpu.SemaphoreType.DMA] * (2 * NBUF),
)
def _agg_kernel(hp_hbm, eif_hbm, dstp_hbm, out_hbm,
                idx_s, idx_d, buf, agg_sh, *sems):
    gsems, ssems = sems[:NBUF], sems[NBUF:]
    c = lax.axis_index("c")
    s = lax.axis_index("s")
    wid = c * NS + s
    pltpu.sync_copy(eif_hbm.at[pl.ds(wid * EPW, EPW)], idx_s.at[pl.ds(0, EPW)])
    for k in range((EPWP - EPW) // 16):
        idx_s[pl.ds(EPW + 16 * k, 16)] = wid * 256 + 16 * k + lax.iota(jnp.int32, 16)

    def zrow(r, carry):
        for k in range(D // 16):
            buf[0, r, pl.ds(k * 16, 16)] = jnp.zeros((16,), jnp.float32)
        return carry

    lax.fori_loop(0, CHUNK, zrow, 0)
    for q in range(RPT // CHUNK):
        pltpu.sync_copy(buf.at[0], agg_sh.at[pl.ds(s * RPT + q * CHUNK, CHUNK)])
    plsc.subcore_barrier()

    def _gwait(b, j):
        pltpu.make_async_copy(hp_hbm.at[idx_s.at[pl.ds(j * CHUNK, CHUNK)]],
                              buf.at[b], gsems[b]).wait()

    def _sstart(b, j):
        pltpu.async_copy(buf.at[b], agg_sh.at[idx_d.at[j]], ssems[b], add=True)

    def _swait(b, j):
        pltpu.make_async_copy(buf.at[b], agg_sh.at[idx_d.at[j]], ssems[b]).wait()

    def _gstart(b, j):
        pltpu.async_copy(hp_hbm.at[idx_s.at[pl.ds(j * CHUNK, CHUNK)]],
                         buf.at[b], gsems[b])

    for p in range(CPT // HPT):
        j0 = p * HPT
        pltpu.sync_copy(dstp_hbm.at[wid, pl.ds(p * HPT, HPT)], idx_d)
        for b in range(NBUF):
            _gstart(b, j0 + b)

        def body(i, carry):
            for b in range(NBUF):
                j = NBUF * i + b
                _gwait(b, j0 + j)
                _sstart(b, j)
            for b in range(NBUF):
                j = NBUF * i + b
                _swait(b, j)
                _gstart(b, j0 + j + NBUF)
            return carry

        lax.fori_loop(0, HPT // NBUF - 1, body, 0)
        for b in range(NBUF):
            j = HPT - NBUF + b
            _gwait(b, j0 + j)
            _sstart(b, j)
        for b in range(NBUF):
            j = HPT - NBUF + b
            _swait(b, j)
    plsc.subcore_barrier()
    pltpu.sync_copy(agg_sh.at[pl.ds(s * RPT, RPT)],
                    out_hbm.at[c, pl.ds(s * RPT, RPT)])


BM = 1000


def _hprime_body(x_ref, w_ref, degc_ref, hp_ref):
    dinv = lax.rsqrt(degc_ref[...] + 1.0)
    hp_ref[...] = jnp.dot(x_ref[...] * dinv, w_ref[...],
                          preferred_element_type=jnp.float32)


def _hprime(x, w, degc):
    return pl.pallas_call(
        _hprime_body,
        grid=(N // BM,),
        in_specs=[
            pl.BlockSpec((BM, D), lambda i: (i, 0)),
            pl.BlockSpec((D, D), lambda i: (0, 0)),
            pl.BlockSpec((BM, 1), lambda i: (i, 0)),
        ],
        out_specs=pl.BlockSpec((BM, D), lambda i: (i, 0)),
        out_shape=jax.ShapeDtypeStruct((N, D), jnp.float32),
    )(x, w, degc)


def _epilogue_body(x_ref, hp_ref, aggp_ref, degc_ref, b_ref, out_ref):
    dinv = lax.rsqrt(degc_ref[...] + 1.0)
    agg = aggp_ref[0] + aggp_ref[1] + hp_ref[...]
    out_ref[...] = x_ref[...] + jnp.maximum(dinv * agg + b_ref[...], 0.0)


def _epilogue(x, hp, aggp, degc, b):
    return pl.pallas_call(
        _epilogue_body,
        grid=(N // BM,),
        in_specs=[
            pl.BlockSpec((BM, D), lambda i: (i, 0)),
            pl.BlockSpec((BM, D), lambda i: (i, 0)),
            pl.BlockSpec((NC, BM, D), lambda i: (0, i, 0)),
            pl.BlockSpec((BM, 1), lambda i: (i, 0)),
            pl.BlockSpec((1, D), lambda i: (0, 0)),
        ],
        out_specs=pl.BlockSpec((BM, D), lambda i: (i, 0)),
        out_shape=jax.ShapeDtypeStruct((N, D), jnp.float32),
    )(x, hp, aggp, degc, b.reshape(1, D))


def kernel(x, edge_index, W, b):
    eif = edge_index.astype(jnp.int32).reshape(2 * E)
    padd = jnp.broadcast_to(N + jnp.arange(EPWP - EPW, dtype=jnp.int32),
                            (NW, EPWP - EPW))
    dstp = jnp.concatenate([lax.slice(eif, (E,), (2 * E,)).reshape(NW, EPW),
                            padd], axis=1)
    dstp = dstp.reshape(NW, CPT, CHUNK)

    degp = _deg_kernel(eif)
    degc = jnp.sum(degp, axis=0).reshape(TR, 1)
    hp = _hprime(x, W, degc)
    aggp = _agg_kernel(hp, eif, dstp)
    return _epilogue(x, hp, aggp, degc, b)


# speedup vs baseline: 35.2422x; 1.0225x over previous
import functools

import jax
import jax.numpy as jnp
from jax import lax
from jax.experimental import pallas as pl
from jax.experimental.pallas import tpu as pltpu
from jax.experimental.pallas import tpu_sc as plsc

N = 10000
E = 320000
D = 128

NC = 2
NS = 16
CHUNK = 128
NW = NC * NS
NBUF = 2
CPT = 80
HPT = CPT // 2
EP = CPT * CHUNK * NW
TR = 10240
RPT = TR // NS

_mesh = plsc.VectorSubcoreMesh(core_axis_name="c", subcore_axis_name="s")


EPW = E // NW
EPWP = CPT * CHUNK


@functools.partial(
    pl.kernel,
    out_type=jax.ShapeDtypeStruct((NW, TR), jnp.float32),
    mesh=_mesh,
    compiler_params=pltpu.CompilerParams(needs_layout_passes=False),
    scratch_types=[
        pltpu.VMEM((EPWP,), jnp.int32),
        pltpu.VMEM((TR,), jnp.float32),
    ],
)
def _deg_kernel(eif_hbm, out_hbm, idx_v, hist):
    c = lax.axis_index("c")
    s = lax.axis_index("s")
    wid = c * NS + s
    pltpu.sync_copy(eif_hbm.at[pl.ds(E + wid * EPW, EPW)], idx_v.at[pl.ds(0, EPW)])
    for k in range((EPWP - EPW) // 16):
        idx_v[pl.ds(EPW + 16 * k, 16)] = N + 16 * k + lax.iota(jnp.int32, 16)

    def zbody(i, carry):
        hist[pl.ds(i * 16, 16)] = jnp.zeros((16,), jnp.float32)
        return carry

    lax.fori_loop(0, TR // 16, zbody, 0)

    def body(j, carry):
        x = idx_v[pl.ds(j * 16, 16)]
        cnt, last = plsc.scan_count(x)
        plsc.addupdate_scatter(hist, [x], cnt.astype(jnp.float32), mask=last)
        return carry

    lax.fori_loop(0, EPWP // 16, body, 0)
    pltpu.sync_copy(hist, out_hbm.at[wid])


@functools.partial(
    pl.kernel,
    out_type=jax.ShapeDtypeStruct((NC, TR, D), jnp.float32),
    mesh=_mesh,
    scratch_types=[
        pltpu.VMEM((EPWP,), jnp.int32),
        pltpu.VMEM((HPT, CHUNK), jnp.int32),
        pltpu.VMEM((NBUF, CHUNK, D), jnp.float32),
        pltpu.VMEM_SHARED((TR, D), jnp.float32),
    ] + [pltpu.SemaphoreType.DMA] * (2 * NBUF),
)
def _agg_kernel(hp_hbm, eif_hbm, dstp_hbm, out_hbm,
                idx_s, idx_d, buf, agg_sh, *sems):
    gsems, ssems = sems[:NBUF], sems[NBUF:]
    c = lax.axis_index("c")
    s = lax.axis_index("s")
    wid = c * NS + s
    pltpu.sync_copy(eif_hbm.at[pl.ds(wid * EPW, EPW)], idx_s.at[pl.ds(0, EPW)])
    for k in range((EPWP - EPW) // 16):
        idx_s[pl.ds(EPW + 16 * k, 16)] = wid * 256 + 16 * k + lax.iota(jnp.int32, 16)

    def zrow(r, carry):
        for k in range(D // 16):
            buf[0, r, pl.ds(k * 16, 16)] = jnp.zeros((16,), jnp.float32)
        return carry

    lax.fori_loop(0, CHUNK, zrow, 0)
    for q in range(RPT // CHUNK):
        pltpu.sync_copy(buf.at[0], agg_sh.at[pl.ds(s * RPT + q * CHUNK, CHUNK)])
    plsc.subcore_barrier()

    def _gwait(b, j):
        pltpu.make_async_copy(hp_hbm.at[idx_s.at[pl.ds(j * CHUNK, CHUNK)]],
                              buf.at[b], gsems[b]).wait()

    def _sstart(b, j):
        pltpu.async_copy(buf.at[b], agg_sh.at[idx_d.at[j]], ssems[b], add=True)

    def _swait(b, j):
        pltpu.make_async_copy(buf.at[b], agg_sh.at[idx_d.at[j]], ssems[b]).wait()

    def _gstart(b, j):
        pltpu.async_copy(hp_hbm.at[idx_s.at[pl.ds(j * CHUNK, CHUNK)]],
                         buf.at[b], gsems[b])

    for p in range(CPT // HPT):
        j0 = p * HPT
        pltpu.sync_copy(dstp_hbm.at[wid, pl.ds(p * HPT, HPT)], idx_d)
        for b in range(NBUF):
            _gstart(b, j0 + b)

        def body(i, carry):
            for b in range(NBUF):
                j = NBUF * i + b
                _gwait(b, j0 + j)
                _sstart(b, j)
            for b in range(NBUF):
                j = NBUF * i + b
                _swait(b, j)
                _gstart(b, j0 + j + NBUF)
            return carry

        lax.fori_loop(0, HPT // NBUF - 1, body, 0)
        for b in range(NBUF):
            j = HPT - NBUF + b
            _gwait(b, j0 + j)
            _sstart(b, j)
        for b in range(NBUF):
            j = HPT - NBUF + b
            _swait(b, j)
    plsc.subcore_barrier()
    pltpu.sync_copy(agg_sh.at[pl.ds(s * RPT, RPT)],
                    out_hbm.at[c, pl.ds(s * RPT, RPT)])


BM = 2000


def _yscale_body(x_ref, degc_ref, y_ref):
    dinv = lax.rsqrt(degc_ref[...] + 1.0)
    y_ref[...] = x_ref[...] * dinv


def _yscale(x, degc):
    return pl.pallas_call(
        _yscale_body,
        grid=(N // BM,),
        in_specs=[
            pl.BlockSpec((BM, D), lambda i: (i, 0)),
            pl.BlockSpec((BM, 1), lambda i: (i, 0)),
        ],
        out_specs=pl.BlockSpec((BM, D), lambda i: (i, 0)),
        out_shape=jax.ShapeDtypeStruct((N, D), jnp.float32),
    )(x, degc)


def _final_body(x_ref, y_ref, aggp_ref, degc_ref, w_ref, b_ref, out_ref):
    dinv = lax.rsqrt(degc_ref[...] + 1.0)
    z = (aggp_ref[0] + aggp_ref[1] + y_ref[...]) * dinv
    h = jnp.dot(z, w_ref[...], preferred_element_type=jnp.float32)
    out_ref[...] = x_ref[...] + jnp.maximum(h + b_ref[...], 0.0)


def _final(x, y, aggp, degc, w, b):
    return pl.pallas_call(
        _final_body,
        grid=(N // BM,),
        in_specs=[
            pl.BlockSpec((BM, D), lambda i: (i, 0)),
            pl.BlockSpec((BM, D), lambda i: (i, 0)),
            pl.BlockSpec((NC, BM, D), lambda i: (0, i, 0)),
            pl.BlockSpec((BM, 1), lambda i: (i, 0)),
            pl.BlockSpec((D, D), lambda i: (0, 0)),
            pl.BlockSpec((1, D), lambda i: (0, 0)),
        ],
        out_specs=pl.BlockSpec((BM, D), lambda i: (i, 0)),
        out_shape=jax.ShapeDtypeStruct((N, D), jnp.float32),
    )(x, y, aggp, degc, w, b.reshape(1, D))


def kernel(x, edge_index, W, b):
    eif = edge_index.astype(jnp.int32).reshape(2 * E)
    padd = jnp.broadcast_to(N + jnp.arange(EPWP - EPW, dtype=jnp.int32),
                            (NW, EPWP - EPW))
    dstp = jnp.concatenate([lax.slice(eif, (E,), (2 * E,)).reshape(NW, EPW),
                            padd], axis=1)
    dstp = dstp.reshape(NW, CPT, CHUNK)

    degp = _deg_kernel(eif)
    degc = jnp.sum(degp[..., None], axis=0)
    y = _yscale(x, degc)
    aggp = _agg_kernel(y, eif, dstp)
    return _final(x, y, aggp, degc, W, b)


# speedup vs baseline: 35.6711x; 1.0122x over previous
import functools

import jax
import jax.numpy as jnp
from jax import lax
from jax.experimental import pallas as pl
from jax.experimental.pallas import tpu as pltpu
from jax.experimental.pallas import tpu_sc as plsc

N = 10000
E = 320000
D = 128

NC = 2
NS = 16
CHUNK = 128
NW = NC * NS
NBUF = 2
CPT = 80
HPT = CPT // 2
EP = CPT * CHUNK * NW
TR = 10240
RPT = TR // NS

_mesh = plsc.VectorSubcoreMesh(core_axis_name="c", subcore_axis_name="s")


EPW = E // NW
EPWP = CPT * CHUNK


@functools.partial(
    pl.kernel,
    out_type=jax.ShapeDtypeStruct((NW, TR), jnp.float32),
    mesh=_mesh,
    compiler_params=pltpu.CompilerParams(needs_layout_passes=False),
    scratch_types=[
        pltpu.VMEM((EPWP,), jnp.int32),
        pltpu.VMEM((TR,), jnp.float32),
    ],
)
def _deg_kernel(eif_hbm, out_hbm, idx_v, hist):
    c = lax.axis_index("c")
    s = lax.axis_index("s")
    wid = c * NS + s
    pltpu.sync_copy(eif_hbm.at[pl.ds(E + wid * EPW, EPW)], idx_v.at[pl.ds(0, EPW)])
    for k in range((EPWP - EPW) // 16):
        idx_v[pl.ds(EPW + 16 * k, 16)] = N + 16 * k + lax.iota(jnp.int32, 16)

    def zbody(i, carry):
        for u in range(8):
            hist[pl.ds(i * 128 + u * 16, 16)] = jnp.zeros((16,), jnp.float32)
        return carry

    lax.fori_loop(0, TR // 128, zbody, 0)

    def body(j, carry):
        for u in range(4):
            x = idx_v[pl.ds(j * 64 + u * 16, 16)]
            cnt, last = plsc.scan_count(x)
            plsc.addupdate_scatter(hist, [x], cnt.astype(jnp.float32), mask=last)
        return carry

    lax.fori_loop(0, EPWP // 64, body, 0)
    pltpu.sync_copy(hist, out_hbm.at[wid])


@functools.partial(
    pl.kernel,
    out_type=jax.ShapeDtypeStruct((NC, TR, D), jnp.float32),
    mesh=_mesh,
    scratch_types=[
        pltpu.VMEM((EPWP,), jnp.int32),
        pltpu.VMEM((HPT, CHUNK), jnp.int32),
        pltpu.VMEM((NBUF, CHUNK, D), jnp.float32),
        pltpu.VMEM_SHARED((TR, D), jnp.float32),
    ] + [pltpu.SemaphoreType.DMA] * (2 * NBUF),
)
def _agg_kernel(hp_hbm, eif_hbm, dstp_hbm, out_hbm,
                idx_s, idx_d, buf, agg_sh, *sems):
    gsems, ssems = sems[:NBUF], sems[NBUF:]
    c = lax.axis_index("c")
    s = lax.axis_index("s")
    wid = c * NS + s
    pltpu.sync_copy(eif_hbm.at[pl.ds(wid * EPW, EPW)], idx_s.at[pl.ds(0, EPW)])
    for k in range((EPWP - EPW) // 16):
        idx_s[pl.ds(EPW + 16 * k, 16)] = wid * 256 + 16 * k + lax.iota(jnp.int32, 16)

    def zrow(r, carry):
        for k in range(D // 16):
            buf[0, r, pl.ds(k * 16, 16)] = jnp.zeros((16,), jnp.float32)
        return carry

    lax.fori_loop(0, CHUNK, zrow, 0)
    for q in range(RPT // CHUNK):
        pltpu.sync_copy(buf.at[0], agg_sh.at[pl.ds(s * RPT + q * CHUNK, CHUNK)])
    plsc.subcore_barrier()

    def _gwait(b, j):
        pltpu.make_async_copy(hp_hbm.at[idx_s.at[pl.ds(j * CHUNK, CHUNK)]],
                              buf.at[b], gsems[b]).wait()

    def _sstart(b, j):
        pltpu.async_copy(buf.at[b], agg_sh.at[idx_d.at[j]], ssems[b], add=True)

    def _swait(b, j):
        pltpu.make_async_copy(buf.at[b], agg_sh.at[idx_d.at[j]], ssems[b]).wait()

    def _gstart(b, j):
        pltpu.async_copy(hp_hbm.at[idx_s.at[pl.ds(j * CHUNK, CHUNK)]],
                         buf.at[b], gsems[b])

    for p in range(CPT // HPT):
        j0 = p * HPT
        pltpu.sync_copy(dstp_hbm.at[wid, pl.ds(p * HPT, HPT)], idx_d)
        for b in range(NBUF):
            _gstart(b, j0 + b)

        def body(i, carry):
            for b in range(NBUF):
                j = NBUF * i + b
                _gwait(b, j0 + j)
                _sstart(b, j)
            for b in range(NBUF):
                j = NBUF * i + b
                _swait(b, j)
                _gstart(b, j0 + j + NBUF)
            return carry

        lax.fori_loop(0, HPT // NBUF - 1, body, 0)
        for b in range(NBUF):
            j = HPT - NBUF + b
            _gwait(b, j0 + j)
            _sstart(b, j)
        for b in range(NBUF):
            j = HPT - NBUF + b
            _swait(b, j)
    plsc.subcore_barrier()
    pltpu.sync_copy(agg_sh.at[pl.ds(s * RPT, RPT)],
                    out_hbm.at[c, pl.ds(s * RPT, RPT)])


BM = 2000


def _yscale_body(x_ref, degc_ref, y_ref):
    dinv = lax.rsqrt(degc_ref[...] + 1.0)
    y_ref[...] = x_ref[...] * dinv


def _yscale(x, degc):
    return pl.pallas_call(
        _yscale_body,
        grid=(N // BM,),
        in_specs=[
            pl.BlockSpec((BM, D), lambda i: (i, 0)),
            pl.BlockSpec((BM, 1), lambda i: (i, 0)),
        ],
        out_specs=pl.BlockSpec((BM, D), lambda i: (i, 0)),
        out_shape=jax.ShapeDtypeStruct((N, D), jnp.float32),
    )(x, degc)


def _final_body(x_ref, y_ref, aggp_ref, degc_ref, w_ref, b_ref, out_ref):
    dinv = lax.rsqrt(degc_ref[...] + 1.0)
    z = (aggp_ref[0] + aggp_ref[1] + y_ref[...]) * dinv
    h = jnp.dot(z, w_ref[...], preferred_element_type=jnp.float32)
    out_ref[...] = x_ref[...] + jnp.maximum(h + b_ref[...], 0.0)


def _final(x, y, aggp, degc, w, b):
    return pl.pallas_call(
        _final_body,
        grid=(N // BM,),
        in_specs=[
            pl.BlockSpec((BM, D), lambda i: (i, 0)),
            pl.BlockSpec((BM, D), lambda i: (i, 0)),
            pl.BlockSpec((NC, BM, D), lambda i: (0, i, 0)),
            pl.BlockSpec((BM, 1), lambda i: (i, 0)),
            pl.BlockSpec((D, D), lambda i: (0, 0)),
            pl.BlockSpec((1, D), lambda i: (0, 0)),
        ],
        out_specs=pl.BlockSpec((BM, D), lambda i: (i, 0)),
        out_shape=jax.ShapeDtypeStruct((N, D), jnp.float32),
    )(x, y, aggp, degc, w, b.reshape(1, D))


def kernel(x, edge_index, W, b):
    eif = edge_index.astype(jnp.int32).reshape(2 * E)
    padd = jnp.broadcast_to(N + jnp.arange(EPWP - EPW, dtype=jnp.int32),
                            (NW, EPWP - EPW))
    dstp = jnp.concatenate([lax.slice(eif, (E,), (2 * E,)).reshape(NW, EPW),
                            padd], axis=1)
    dstp = dstp.reshape(NW, CPT, CHUNK)

    degp = _deg_kernel(eif)
    degc = jnp.sum(degp[..., None], axis=0)
    y = _yscale(x, degc)
    aggp = _agg_kernel(y, eif, dstp)
    return _final(x, y, aggp, degc, W, b)
